# baseline (speedup 1.0000x reference)
import functools

import jax
import jax.numpy as jnp
from jax import lax
from jax.experimental import pallas as pl
from jax.experimental.pallas import tpu as pltpu
from jax.experimental.pallas import tpu_sc as plsc

NUM_STEPS = 1000
HIDDEN = 64
WIDE = 128
BATCH = 16384
PAD_STEPS = 1024


def _mlp_table_kernel(t_ref, w1_ref, b1_ref, w2_ref, b2_ref, o_ref):
    h = jnp.dot(t_ref[...], w1_ref[...], preferred_element_type=jnp.float32)
    h = h + b1_ref[...]
    h = h * jax.nn.sigmoid(h)
    o = jnp.dot(h, w2_ref[...], preferred_element_type=jnp.float32)
    o = o + b2_ref[...]
    o_ref[0:NUM_STEPS, :] = o


def _mlp_table(table, W1, b1, W2, b2):
    return pl.pallas_call(
        _mlp_table_kernel,
        out_shape=jax.ShapeDtypeStruct((PAD_STEPS, HIDDEN), jnp.float32),
    )(table, W1, b1.reshape(1, -1), W2, b2.reshape(1, -1))


def _make_sc_gather(nc, ns):
    nw = nc * ns
    rows_per_w = BATCH // nw
    chunks = rows_per_w // 128
    mesh = plsc.VectorSubcoreMesh(core_axis_name="c", subcore_axis_name="s")

    @functools.partial(
        pl.kernel,
        mesh=mesh,
        out_type=jax.ShapeDtypeStruct((nw * chunks, 128, HIDDEN), jnp.float32),
        scratch_types=[
            pltpu.VMEM((chunks, 128), jnp.int32),
            pltpu.VMEM((chunks, 128, HIDDEN), jnp.float32),
            pltpu.SemaphoreType.DMA,
        ],
        compiler_params=pltpu.CompilerParams(use_tc_tiling_on_sc=False),
    )
    def gather(table_hbm, idx_hbm, out_hbm, idx_v, rows_v, sem):
        wid = lax.axis_index("s") * nc + lax.axis_index("c")
        base = wid * chunks
        pltpu.sync_copy(idx_hbm.at[pl.ds(base, chunks)], idx_v)
        copies = []
        for j in range(chunks):
            copies.append(
                pltpu.async_copy(table_hbm.at[idx_v.at[j]], rows_v.at[j], sem)
            )
        for c in copies:
            c.wait()
        pltpu.sync_copy(rows_v, out_hbm.at[pl.ds(base, chunks)])

    return gather


def kernel(timesteps, emb_table, W1, b1, W2, b2):
    out_table = _mlp_table(emb_table, W1, b1, W2, b2)

    info = plsc.get_sparse_core_info()
    nc, ns = info.num_cores, info.num_subcores
    idx = timesteps.astype(jnp.int32).reshape(BATCH // 128, 128)
    out = _make_sc_gather(nc, ns)(out_table, idx)
    return out.reshape(BATCH, HIDDEN)


# speedup vs baseline: 1.1538x; 1.1538x over previous
import functools

import jax
import jax.numpy as jnp
from jax import lax
from jax.experimental import pallas as pl
from jax.experimental.pallas import tpu as pltpu
from jax.experimental.pallas import tpu_sc as plsc

NUM_STEPS = 1000
HIDDEN = 64
WIDE = 128
BATCH = 16384
PAD_STEPS = 1024


def _mlp_table_kernel(t_ref, w1_ref, b1_ref, w2_ref, b2_ref, o_ref):
    h = jnp.dot(t_ref[...], w1_ref[...], preferred_element_type=jnp.float32)
    h = h + b1_ref[...]
    h = h * jax.nn.sigmoid(h)
    o = jnp.dot(h, w2_ref[...], preferred_element_type=jnp.float32)
    o = o + b2_ref[...]
    o_ref[0:NUM_STEPS, 0:HIDDEN] = o


def _mlp_table(table, W1, b1, W2, b2):
    return pl.pallas_call(
        _mlp_table_kernel,
        out_shape=jax.ShapeDtypeStruct((PAD_STEPS, WIDE), jnp.float32),
    )(table, W1, b1.reshape(1, -1), W2, b2.reshape(1, -1))


def _make_sc_gather(nc, ns):
    nw = nc * ns
    rows_per_w = BATCH // nw
    chunks = rows_per_w // 128
    mesh = plsc.VectorSubcoreMesh(core_axis_name="c", subcore_axis_name="s")

    @functools.partial(
        pl.kernel,
        mesh=mesh,
        out_type=jax.ShapeDtypeStruct((nw * chunks, 128, WIDE), jnp.float32),
        scratch_types=[
            pltpu.VMEM((chunks, 128), jnp.int32),
            pltpu.VMEM((chunks, 128, WIDE), jnp.float32),
            pltpu.SemaphoreType.DMA,
        ],
    )
    def gather(table_hbm, idx_hbm, out_hbm, idx_v, rows_v, sem):
        wid = lax.axis_index("s") * nc + lax.axis_index("c")
        base = wid * chunks
        pltpu.sync_copy(idx_hbm.at[pl.ds(base, chunks)], idx_v)
        copies = []
        for j in range(chunks):
            copies.append(
                pltpu.async_copy(table_hbm.at[idx_v.at[j]], rows_v.at[j], sem)
            )
        for c in copies:
            c.wait()
        pltpu.sync_copy(rows_v, out_hbm.at[pl.ds(base, chunks)])

    return gather


def kernel(timesteps, emb_table, W1, b1, W2, b2):
    out_table = _mlp_table(emb_table, W1, b1, W2, b2)

    info = plsc.get_sparse_core_info()
    nc, ns = info.num_cores, info.num_subcores
    idx = timesteps.astype(jnp.int32).reshape(BATCH // 128, 128)
    out = _make_sc_gather(nc, ns)(out_table, idx)
    return out.reshape(BATCH, WIDE)[:, :HIDDEN]
